# baseline (speedup 1.0000x reference)
import functools

import jax
import jax.numpy as jnp
from jax import lax
from jax.experimental import pallas as pl
from jax.experimental.pallas import tpu as pltpu
from jax.experimental.pallas import tpu_sc as plsc

NUM_FACTORS = 32
BATCH = 16384
NUM_CORES = 2
NUM_SUBCORES = 16
NUM_WORKERS = NUM_CORES * NUM_SUBCORES
BPW = BATCH // NUM_WORKERS
LANES = 16
GROUPS = BPW // LANES

_mesh = plsc.VectorSubcoreMesh(core_axis_name="c", subcore_axis_name="s")


@functools.partial(
    pl.kernel,
    out_type=jax.ShapeDtypeStruct((BATCH,), jnp.float32),
    mesh=_mesh,
    scratch_types=[
        pltpu.VMEM((BPW,), jnp.int32),
        pltpu.VMEM((BPW,), jnp.int32),
        pltpu.VMEM((BPW,), jnp.int32),
        pltpu.VMEM((BPW, NUM_FACTORS), jnp.float32),
        pltpu.VMEM((BPW, NUM_FACTORS), jnp.float32),
        pltpu.VMEM((BPW, NUM_FACTORS), jnp.float32),
        pltpu.VMEM((BPW,), jnp.float32),
        pltpu.VMEM((BPW,), jnp.float32),
        pltpu.VMEM((BPW,), jnp.float32),
        pltpu.VMEM((LANES,), jnp.float32),
        pltpu.VMEM((BPW,), jnp.float32),
        pltpu.VMEM((LANES * LANES,), jnp.float32),
        pltpu.SemaphoreType.DMA,
    ],
    compiler_params=pltpu.CompilerParams(
        needs_layout_passes=False, use_tc_tiling_on_sc=False
    ),
)
def _mf_sc(inv_i, tic_i, dat_i, inv_f, tic_f, dat_f, inv_b, tic_b, dat_b,
           gb, out, idx_inv, idx_tic, idx_dat, rows_inv, rows_tic,
           rows_dat, bias_inv, bias_tic, bias_dat, gb_v, out_v, part_v, sem):
    wid = lax.axis_index("s") * NUM_CORES + lax.axis_index("c")
    base = wid * BPW

    pltpu.sync_copy(inv_i.at[pl.ds(base, BPW)], idx_inv)
    pltpu.sync_copy(tic_i.at[pl.ds(base, BPW)], idx_tic)
    pltpu.sync_copy(dat_i.at[pl.ds(base, BPW)], idx_dat)
    pltpu.sync_copy(gb, gb_v)

    copies = [
        pltpu.async_copy(inv_f.at[idx_inv], rows_inv, sem),
        pltpu.async_copy(tic_f.at[idx_tic], rows_tic, sem),
        pltpu.async_copy(dat_f.at[idx_dat], rows_dat, sem),
        pltpu.async_copy(inv_b.at[idx_inv], bias_inv, sem),
        pltpu.async_copy(tic_b.at[idx_tic], bias_tic, sem),
        pltpu.async_copy(dat_b.at[idx_dat], bias_dat, sem),
    ]
    for cp in copies:
        cp.wait()

    lanes = lax.iota(jnp.int32, 16)
    gbv = gb_v[...]

    def group(g, carry):
        o = g * LANES
        for j in range(LANES):
            row = o + j
            a_lo = rows_inv[row, pl.ds(0, LANES)]
            a_hi = rows_inv[row, pl.ds(LANES, LANES)]
            t_lo = rows_tic[row, pl.ds(0, LANES)]
            t_hi = rows_tic[row, pl.ds(LANES, LANES)]
            d_lo = rows_dat[row, pl.ds(0, LANES)]
            d_hi = rows_dat[row, pl.ds(LANES, LANES)]
            part_v[pl.ds(j * LANES, LANES)] = a_lo * t_lo * d_lo + a_hi * t_hi * d_hi
        acc = bias_inv[pl.ds(o, LANES)] + bias_tic[pl.ds(o, LANES)]
        acc = acc + bias_dat[pl.ds(o, LANES)] + gbv
        rbase = lanes * LANES
        for c in range(LANES):
            acc = acc + plsc.load_gather(part_v, [rbase + c])
        out_v[pl.ds(o, LANES)] = acc
        return carry

    lax.fori_loop(0, GROUPS, group, 0)
    pltpu.sync_copy(out_v, out.at[pl.ds(base, BPW)])


def kernel(investor, ticker, date, investor_factors, ticker_factors,
           date_factors, investor_bias, ticker_bias, date_bias, global_bias):
    gb16 = jnp.broadcast_to(global_bias.astype(jnp.float32), (LANES,))
    return _mf_sc(
        investor.astype(jnp.int32),
        ticker.astype(jnp.int32),
        date.astype(jnp.int32),
        investor_factors,
        ticker_factors,
        date_factors,
        investor_bias.reshape(-1),
        ticker_bias.reshape(-1),
        date_bias.reshape(-1),
        gb16,
    )
